# Initial kernel scaffold; baseline (speedup 1.0000x reference)
import jax
import jax.numpy as jnp
from jax.experimental import pallas as pl


def kernel(x, edge_index, edge_attr, batch, W1s, W1n, b1, W2s, W2n, b2, We, be):
    raise NotImplementedError("write your pallas kernel here")


# speedup vs baseline: 12.3958x; 12.3958x over previous
import functools

import jax
import jax.numpy as jnp
from jax import lax
from jax.experimental import pallas as pl
from jax.experimental.pallas import tpu as pltpu
from jax.experimental.pallas import tpu_sc as plsc

NN = 10000
EE = 320000
DD = 128
HH = 16
GG = 64

NC = 2
NS = 16
NW = NC * NS
CH = 128
KC = 79
EP = NW * KC * CH
NP = 10112
RT = NP // NS

_f32 = jnp.float32


def _edge_agg_body(with_deg, *refs):
    if with_deg:
        (table_h, src_h, dst_h, acc_out, deg_out,
         srcv, dstv, rows, zst, ones, acc_sh, deg_sh, sem) = refs
    else:
        (table_h, src_h, dst_h, acc_out,
         srcv, dstv, rows, zst, acc_sh, sem) = refs

    cid = lax.axis_index("c")
    sid = lax.axis_index("s")
    wid = sid * NC + cid

    zero16 = jnp.zeros((HH,), _f32)

    def zb(i, carry):
        zst[i, :] = zero16
        return carry

    lax.fori_loop(0, RT, zb, 0)

    if with_deg:
        one16 = jnp.ones((HH,), _f32)

        def ob(i, carry):
            ones[i, :] = one16
            return carry

        lax.fori_loop(0, CH, ob, 0)

    pltpu.sync_copy(zst, acc_sh.at[pl.ds(sid * RT, RT)])
    if with_deg:
        pltpu.sync_copy(zst, deg_sh.at[pl.ds(sid * RT, RT)])

    pltpu.sync_copy(src_h.at[wid], srcv)
    pltpu.sync_copy(dst_h.at[wid], dstv)

    plsc.subcore_barrier()

    def body(j, carry):
        pltpu.async_copy(table_h.at[srcv.at[j]], rows, sem).wait()
        pltpu.sync_copy(rows, acc_sh.at[dstv.at[j]], add=True)
        if with_deg:
            pltpu.sync_copy(ones, deg_sh.at[dstv.at[j]], add=True)
        return carry

    lax.fori_loop(0, KC, body, 0)

    plsc.subcore_barrier()

    pltpu.sync_copy(acc_sh.at[pl.ds(sid * RT, RT)],
                    acc_out.at[cid, pl.ds(sid * RT, RT)])
    if with_deg:
        pltpu.sync_copy(deg_sh.at[pl.ds(sid * RT, RT)],
                        deg_out.at[cid, pl.ds(sid * RT, RT)])


def _edge_agg(table, srcp, dstp, with_deg):
    out_type = [jax.ShapeDtypeStruct((NC, NP, HH), _f32)]
    scratch = [
        pltpu.VMEM((KC, CH), jnp.int32),
        pltpu.VMEM((KC, CH), jnp.int32),
        pltpu.VMEM((CH, HH), _f32),
        pltpu.VMEM((RT, HH), _f32),
    ]
    if with_deg:
        out_type.append(jax.ShapeDtypeStruct((NC, NP, HH), _f32))
        scratch.append(pltpu.VMEM((CH, HH), _f32))
    scratch.append(pltpu.VMEM_SHARED((NP, HH), _f32))
    if with_deg:
        scratch.append(pltpu.VMEM_SHARED((NP, HH), _f32))
    scratch.append(pltpu.SemaphoreType.DMA)

    mesh = plsc.VectorSubcoreMesh(core_axis_name="c", subcore_axis_name="s")
    fn = pl.kernel(
        functools.partial(_edge_agg_body, with_deg),
        out_type=tuple(out_type),
        mesh=mesh,
        scratch_types=tuple(scratch),
        compiler_params=pltpu.CompilerParams(use_tc_tiling_on_sc=False),
    )
    return fn(table, srcp, dstp)


def _proj_in_body(x_ref, w_ref, b_ref, o_ref):
    o_ref[...] = (
        jnp.dot(x_ref[...], w_ref[...], preferred_element_type=_f32)
        + b_ref[...]
    )


def _mid_body(a0, a1, d0, d1, ys, w, b, o, degc):
    deg = jnp.maximum(d0[...] + d1[...], 1.0)
    h1 = jnp.maximum((a0[...] + a1[...]) / deg + ys[...], 0.0)
    o[...] = jnp.dot(h1, w[...], preferred_element_type=_f32) + b[...]
    degc[...] = deg


def _final_body(a0, a1, degc, ys, wet, bet, bat, o):
    h2 = jnp.maximum((a0[...] + a1[...]) / degc[...] + ys[...], 0.0)
    z = jnp.sum(h2 * wet[...], axis=1)[None, :] + bet[...]
    gid = lax.broadcasted_iota(jnp.int32, (GG, NN), 0)
    m = (gid == bat[...]).astype(_f32)
    s = jnp.sum(m * z, axis=1)
    c = jnp.sum(m, axis=1)
    o[...] = jax.nn.sigmoid(s / jnp.maximum(c, 1.0))[:, None]


def kernel(x, edge_index, edge_attr, batch, W1s, W1n, b1, W2s, W2n, b2,
           We, be):
    del edge_attr

    pad = EP - EE
    src = jnp.concatenate(
        [edge_index[0], jnp.full((pad,), NN, jnp.int32)]).reshape(NW, KC, CH)
    dst = jnp.concatenate(
        [edge_index[1], jnp.full((pad,), NN, jnp.int32)]).reshape(NW, KC, CH)

    wcat1 = jnp.concatenate([W1n, W1s], axis=1)
    bcat1 = jnp.concatenate([jnp.zeros((HH,), _f32), b1]).reshape(1, 2 * HH)
    ycat1 = pl.pallas_call(
        _proj_in_body,
        out_shape=jax.ShapeDtypeStruct((NN, 2 * HH), _f32),
    )(x, wcat1, bcat1)
    y1n = jnp.pad(ycat1[:, :HH], ((0, NP - NN), (0, 0)))
    ys1 = ycat1[:, HH:]

    agg1, deg1 = _edge_agg(y1n, src, dst, with_deg=True)

    wcat2 = jnp.concatenate([W2n, W2s], axis=1)
    bcat2 = jnp.concatenate([jnp.zeros((HH,), _f32), b2]).reshape(1, 2 * HH)
    ycat2, degc = pl.pallas_call(
        _mid_body,
        out_shape=(jax.ShapeDtypeStruct((NN, 2 * HH), _f32),
                   jax.ShapeDtypeStruct((NN, HH), _f32)),
    )(agg1[0, :NN], agg1[1, :NN], deg1[0, :NN], deg1[1, :NN], ys1,
      wcat2, bcat2)
    y2n = jnp.pad(ycat2[:, :HH], ((0, NP - NN), (0, 0)))
    ys2 = ycat2[:, HH:]

    (agg2,) = _edge_agg(y2n, src, dst, with_deg=False)

    wet = We.reshape(1, HH)
    bet = be.reshape(1, 1)
    bat = batch.reshape(1, NN)
    out = pl.pallas_call(
        _final_body,
        out_shape=jax.ShapeDtypeStruct((GG, 1), _f32),
    )(agg2[0, :NN], agg2[1, :NN], degc, ys2, wet, bet, bat)
    return out


# speedup vs baseline: 23.6655x; 1.9092x over previous
import functools

import jax
import jax.numpy as jnp
from jax import lax
from jax.experimental import pallas as pl
from jax.experimental.pallas import tpu as pltpu
from jax.experimental.pallas import tpu_sc as plsc

NN = 10000
EE = 320000
DD = 128
HH = 16
GG = 64

NC = 2
NS = 16
NW = NC * NS
CH = 128
RB = 8
GA = 4
KC = 80
NI = KC // RB
EP = NW * KC * CH
NP = 10112
RT = NP // NS
RP = NP // 8

_f32 = jnp.float32


def _edge_agg_body(with_deg, *refs):
    if with_deg:
        (table_h, ei_h, acc_out, deg_out,
         srcv, dstv, r0, r1, r2, r3, r4, r5, r6, r7, zst, ones,
         acc_sh, deg_sh,
         g0, g1, g2, g3, s0, s1, s2, s3, d0, d1, d2, d3) = refs
    else:
        (table_h, ei_h, acc_out,
         srcv, dstv, r0, r1, r2, r3, r4, r5, r6, r7, zst,
         acc_sh, g0, g1, g2, g3, s0, s1, s2, s3) = refs
        d0 = d1 = d2 = d3 = ones = None

    rows = (r0, r1, r2, r3, r4, r5, r6, r7)
    gsem = (g0, g1, g2, g3)
    ssem = (s0, s1, s2, s3)
    dsem = (d0, d1, d2, d3)

    cid = lax.axis_index("c")
    sid = lax.axis_index("s")
    wid = sid * NC + cid

    zero16 = jnp.zeros((HH,), _f32)

    def zb(i, carry):
        zst[i, :] = zero16
        return carry

    lax.fori_loop(0, RT, zb, 0)

    if with_deg:
        one16 = jnp.ones((HH,), _f32)

        def ob(i, carry):
            ones[i, :] = one16
            return carry

        lax.fori_loop(0, CH, ob, 0)

    pltpu.sync_copy(zst, acc_sh.at[pl.ds(sid * RT, RT)])
    if with_deg:
        pltpu.sync_copy(zst, deg_sh.at[pl.ds(sid * RT, RT)])

    pltpu.sync_copy(ei_h.at[0, wid], srcv)
    pltpu.sync_copy(ei_h.at[1, wid], dstv)

    plsc.subcore_barrier()

    def _drain(buf, sem):
        pltpu.make_async_copy(table_h.at[srcv.at[0]], buf, sem).wait()

    for p in range(GA):
        pltpu.async_copy(table_h.at[srcv.at[p]], rows[p], gsem[p])

    def body(i, carry):
        for r in range(RB):
            j = RB * i + r
            p = r % GA
            pltpu.make_async_copy(
                table_h.at[srcv.at[0]], rows[r], gsem[p]).wait()
            if r < GA:
                @pl.when(i > 0)
                def _():
                    _drain(rows[(r + GA) % RB], ssem[p])
                    if with_deg:
                        _drain(rows[(r + GA) % RB], dsem[p])
            else:
                _drain(rows[(r + GA) % RB], ssem[p])
                if with_deg:
                    _drain(rows[(r + GA) % RB], dsem[p])
            if r < GA:
                pltpu.async_copy(
                    table_h.at[srcv.at[j + GA]], rows[(r + GA) % RB],
                    gsem[p])
            else:
                @pl.when(i < NI - 1)
                def _():
                    pltpu.async_copy(
                        table_h.at[srcv.at[j + GA]], rows[(r + GA) % RB],
                        gsem[p])
            pltpu.async_copy(rows[r], acc_sh.at[dstv.at[j]], ssem[p],
                             add=True)
            if with_deg:
                pltpu.async_copy(ones, deg_sh.at[dstv.at[j]], dsem[p],
                                 add=True)
        return carry

    lax.fori_loop(0, NI, body, 0)

    for p in range(GA):
        _drain(rows[GA + p], ssem[p])
        if with_deg:
            _drain(rows[GA + p], dsem[p])

    plsc.subcore_barrier()

    pltpu.sync_copy(acc_sh.at[pl.ds(sid * RT, RT)],
                    acc_out.at[cid, pl.ds(sid * RT, RT)])
    if with_deg:
        pltpu.sync_copy(deg_sh.at[pl.ds(sid * RT, RT)],
                        deg_out.at[cid, pl.ds(sid * RT, RT)])


def _edge_agg(table, eip, with_deg):
    out_type = [jax.ShapeDtypeStruct((NC, NP, HH), _f32)]
    scratch = [
        pltpu.VMEM((KC, CH), jnp.int32),
        pltpu.VMEM((KC, CH), jnp.int32),
    ]
    scratch += [pltpu.VMEM((CH, HH), _f32)] * RB
    scratch.append(pltpu.VMEM((RT, HH), _f32))
    if with_deg:
        out_type.append(jax.ShapeDtypeStruct((NC, NP, HH), _f32))
        scratch.append(pltpu.VMEM((CH, HH), _f32))
    scratch.append(pltpu.VMEM_SHARED((NP, HH), _f32))
    if with_deg:
        scratch.append(pltpu.VMEM_SHARED((NP, HH), _f32))
    scratch.extend([pltpu.SemaphoreType.DMA] * (3 * GA if with_deg
                                                else 2 * GA))

    mesh = plsc.VectorSubcoreMesh(core_axis_name="c", subcore_axis_name="s")
    fn = pl.kernel(
        functools.partial(_edge_agg_body, with_deg),
        out_type=tuple(out_type),
        mesh=mesh,
        scratch_types=tuple(scratch),
        compiler_params=pltpu.CompilerParams(use_tc_tiling_on_sc=False),
    )
    return fn(table, eip)


def _proj_body(x_ref, wn_ref, ws_ref, b_ref, on_ref, os_ref):
    xv = x_ref[...]
    on_ref[...] = jnp.dot(xv, wn_ref[...], preferred_element_type=_f32)
    os_ref[...] = (jnp.dot(xv, ws_ref[...], preferred_element_type=_f32)
                   + b_ref[...])


def _mid_body(a_ref, d_ref, ys_ref, wn_ref, ws_ref, b_ref,
              on_ref, os_ref, dc_ref):
    deg = jnp.maximum(d_ref[0] + d_ref[1], 1.0)
    h1 = jnp.maximum((a_ref[0] + a_ref[1]) / deg + ys_ref[...], 0.0)
    on_ref[...] = jnp.dot(h1, wn_ref[...], preferred_element_type=_f32)
    os_ref[...] = (jnp.dot(h1, ws_ref[...], preferred_element_type=_f32)
                   + b_ref[...])
    dc_ref[...] = deg


def _final_body(a_ref, dc_ref, ys_ref, wet_ref, gs_ref, bet_ref, bat_ref,
                o_ref):
    h2 = jnp.maximum((a_ref[0] + a_ref[1]) / dc_ref[...] + ys_ref[...],
                     0.0)
    zw = h2 * wet_ref[...]
    zt = lax.dot_general(gs_ref[...], zw, (((0,), (1,)), ((), ())),
                         preferred_element_type=_f32)
    gid = lax.broadcasted_iota(jnp.int32, (GG, 8, RP), 0)
    m = (gid == bat_ref[...][None, :, :]).astype(_f32)
    s = jnp.sum(m * zt[None, :, :], axis=(1, 2))
    c = jnp.sum(m, axis=(1, 2))
    pooled = (s / jnp.maximum(c, 1.0))[:, None]
    pooled = pooled + jnp.where(c[:, None] > 0.0, bet_ref[...], 0.0)
    o_ref[...] = jax.nn.sigmoid(pooled)


def kernel(x, edge_index, edge_attr, batch, W1s, W1n, b1, W2s, W2n, b2,
           We, be):
    del edge_attr

    eip = jnp.pad(edge_index, ((0, 0), (0, EP - EE)),
                  constant_values=NN).reshape(2, NW, KC, CH)

    eye8 = jnp.eye(8, dtype=_f32)
    w1n_bd = jnp.kron(eye8, W1n)
    w1s_bd = jnp.kron(eye8, W1s)
    w2n_bd = jnp.kron(eye8, W2n)
    w2s_bd = jnp.kron(eye8, W2s)
    b1t = jnp.tile(b1, 8).reshape(1, 128)
    b2t = jnp.tile(b2, 8).reshape(1, 128)
    wet = jnp.tile(We[:, 0], 8).reshape(1, 128)
    gs = jnp.kron(jnp.eye(8, dtype=_f32), jnp.ones((HH, 1), _f32))
    bet = be.reshape(1, 1)
    batp = jnp.pad(batch, (0, NP - NN),
                   constant_values=GG).reshape(RP, 8).T

    xp = jnp.pad(x, ((0, NP - NN), (0, 0))).reshape(RP, 8 * DD)
    y1n_p, ys1_p = pl.pallas_call(
        _proj_body,
        out_shape=(jax.ShapeDtypeStruct((RP, 128), _f32),
                   jax.ShapeDtypeStruct((RP, 128), _f32)),
    )(xp, w1n_bd, w1s_bd, b1t)

    agg1, deg1 = _edge_agg(y1n_p.reshape(NP, HH), eip, with_deg=True)

    y2n_p, ys2_p, degc_p = pl.pallas_call(
        _mid_body,
        out_shape=(jax.ShapeDtypeStruct((RP, 128), _f32),
                   jax.ShapeDtypeStruct((RP, 128), _f32),
                   jax.ShapeDtypeStruct((RP, 128), _f32)),
    )(agg1.reshape(NC, RP, 128), deg1.reshape(NC, RP, 128), ys1_p,
      w2n_bd, w2s_bd, b2t)

    (agg2,) = _edge_agg(y2n_p.reshape(NP, HH), eip, with_deg=False)

    out = pl.pallas_call(
        _final_body,
        out_shape=jax.ShapeDtypeStruct((GG, 1), _f32),
    )(agg2.reshape(NC, RP, 128), degc_p, ys2_p, wet, gs, bet, batp)
    return out


# speedup vs baseline: 33.8116x; 1.4287x over previous
import functools

import jax
import jax.numpy as jnp
from jax import lax
from jax.experimental import pallas as pl
from jax.experimental.pallas import tpu as pltpu
from jax.experimental.pallas import tpu_sc as plsc

NN = 10000
EE = 320000
DD = 128
HH = 16
GG = 64

NC = 2
NS = 16
NW = NC * NS
CH = 128
RB = 8
GA = 4
KC = 80
NI = KC // RB
EP = NW * KC * CH
NP = 10112
RT = NP // NS
RP = NP // 8

_f32 = jnp.float32


def _edge_agg_body(with_deg, *refs):
    if with_deg:
        (table_h, ei_h, acc_out, deg_out,
         srcv, dstv, r0, r1, r2, r3, r4, r5, r6, r7, zst, ones,
         table_sh, acc_sh, deg_sh,
         g0, g1, g2, g3, s0, s1, s2, s3, d0, d1, d2, d3) = refs
    else:
        (table_h, ei_h, acc_out,
         srcv, dstv, r0, r1, r2, r3, r4, r5, r6, r7, zst,
         table_sh, acc_sh, g0, g1, g2, g3, s0, s1, s2, s3) = refs
        d0 = d1 = d2 = d3 = ones = None

    rows = (r0, r1, r2, r3, r4, r5, r6, r7)
    gsem = (g0, g1, g2, g3)
    ssem = (s0, s1, s2, s3)
    dsem = (d0, d1, d2, d3)

    cid = lax.axis_index("c")
    sid = lax.axis_index("s")
    wid = sid * NC + cid

    zero16 = jnp.zeros((HH,), _f32)

    def zb(i, carry):
        zst[i, :] = zero16
        return carry

    lax.fori_loop(0, RT, zb, 0)

    if with_deg:
        one16 = jnp.ones((HH,), _f32)

        def ob(i, carry):
            ones[i, :] = one16
            return carry

        lax.fori_loop(0, CH, ob, 0)

    pltpu.sync_copy(zst, acc_sh.at[pl.ds(sid * RT, RT)])
    if with_deg:
        pltpu.sync_copy(zst, deg_sh.at[pl.ds(sid * RT, RT)])

    pltpu.sync_copy(table_h.at[pl.ds(sid * RT, RT)],
                    table_sh.at[pl.ds(sid * RT, RT)])

    pltpu.sync_copy(ei_h.at[0, wid], srcv)
    pltpu.sync_copy(ei_h.at[1, wid], dstv)

    plsc.subcore_barrier()

    def _drain(buf, sem):
        pltpu.make_async_copy(table_h.at[srcv.at[0]], buf, sem).wait()

    for p in range(GA):
        pltpu.async_copy(table_sh.at[srcv.at[p]], rows[p], gsem[p])

    def body(i, carry):
        for r in range(RB):
            j = RB * i + r
            p = r % GA
            pltpu.make_async_copy(
                table_h.at[srcv.at[0]], rows[r], gsem[p]).wait()
            if r < GA:
                @pl.when(i > 0)
                def _():
                    _drain(rows[(r + GA) % RB], ssem[p])
                    if with_deg:
                        _drain(rows[(r + GA) % RB], dsem[p])
            else:
                _drain(rows[(r + GA) % RB], ssem[p])
                if with_deg:
                    _drain(rows[(r + GA) % RB], dsem[p])
            if r < GA:
                pltpu.async_copy(
                    table_sh.at[srcv.at[j + GA]], rows[(r + GA) % RB],
                    gsem[p])
            else:
                @pl.when(i < NI - 1)
                def _():
                    pltpu.async_copy(
                        table_sh.at[srcv.at[j + GA]], rows[(r + GA) % RB],
                        gsem[p])
            pltpu.async_copy(rows[r], acc_sh.at[dstv.at[j]], ssem[p],
                             add=True)
            if with_deg:
                pltpu.async_copy(ones, deg_sh.at[dstv.at[j]], dsem[p],
                                 add=True)
        return carry

    lax.fori_loop(0, NI, body, 0)

    for p in range(GA):
        _drain(rows[GA + p], ssem[p])
        if with_deg:
            _drain(rows[GA + p], dsem[p])

    plsc.subcore_barrier()

    pltpu.sync_copy(acc_sh.at[pl.ds(sid * RT, RT)],
                    acc_out.at[cid, pl.ds(sid * RT, RT)])
    if with_deg:
        pltpu.sync_copy(deg_sh.at[pl.ds(sid * RT, RT)],
                        deg_out.at[cid, pl.ds(sid * RT, RT)])


def _edge_agg(table, eip, with_deg):
    out_type = [jax.ShapeDtypeStruct((NC, NP, HH), _f32)]
    scratch = [
        pltpu.VMEM((KC, CH), jnp.int32),
        pltpu.VMEM((KC, CH), jnp.int32),
    ]
    scratch += [pltpu.VMEM((CH, HH), _f32)] * RB
    scratch.append(pltpu.VMEM((RT, HH), _f32))
    if with_deg:
        out_type.append(jax.ShapeDtypeStruct((NC, NP, HH), _f32))
        scratch.append(pltpu.VMEM((CH, HH), _f32))
    scratch.append(pltpu.VMEM_SHARED((NP, HH), _f32))
    scratch.append(pltpu.VMEM_SHARED((NP, HH), _f32))
    if with_deg:
        scratch.append(pltpu.VMEM_SHARED((NP, HH), _f32))
    scratch.extend([pltpu.SemaphoreType.DMA] * (3 * GA if with_deg
                                                else 2 * GA))

    mesh = plsc.VectorSubcoreMesh(core_axis_name="c", subcore_axis_name="s")
    fn = pl.kernel(
        functools.partial(_edge_agg_body, with_deg),
        out_type=tuple(out_type),
        mesh=mesh,
        scratch_types=tuple(scratch),
        compiler_params=pltpu.CompilerParams(use_tc_tiling_on_sc=False),
    )
    return fn(table, eip)


def _proj_body(x_ref, wn_ref, ws_ref, b_ref, on_ref, os_ref):
    xv = x_ref[...]
    on_ref[...] = jnp.dot(xv, wn_ref[...], preferred_element_type=_f32)
    os_ref[...] = (jnp.dot(xv, ws_ref[...], preferred_element_type=_f32)
                   + b_ref[...])


def _mid_body(a_ref, d_ref, ys_ref, wn_ref, ws_ref, b_ref,
              on_ref, os_ref, dc_ref):
    deg = jnp.maximum(d_ref[0] + d_ref[1], 1.0)
    h1 = jnp.maximum((a_ref[0] + a_ref[1]) / deg + ys_ref[...], 0.0)
    on_ref[...] = jnp.dot(h1, wn_ref[...], preferred_element_type=_f32)
    os_ref[...] = (jnp.dot(h1, ws_ref[...], preferred_element_type=_f32)
                   + b_ref[...])
    dc_ref[...] = deg


def _final_body(a_ref, dc_ref, ys_ref, wet_ref, gs_ref, bet_ref, bat_ref,
                o_ref):
    h2 = jnp.maximum((a_ref[0] + a_ref[1]) / dc_ref[...] + ys_ref[...],
                     0.0)
    zw = h2 * wet_ref[...]
    zt = lax.dot_general(gs_ref[...], zw, (((0,), (1,)), ((), ())),
                         preferred_element_type=_f32)
    gid = lax.broadcasted_iota(jnp.int32, (GG, 8, RP), 0)
    m = (gid == bat_ref[...][None, :, :]).astype(_f32)
    s = jnp.sum(m * zt[None, :, :], axis=(1, 2))
    c = jnp.sum(m, axis=(1, 2))
    pooled = (s / jnp.maximum(c, 1.0))[:, None]
    pooled = pooled + jnp.where(c[:, None] > 0.0, bet_ref[...], 0.0)
    o_ref[...] = jax.nn.sigmoid(pooled)


def kernel(x, edge_index, edge_attr, batch, W1s, W1n, b1, W2s, W2n, b2,
           We, be):
    del edge_attr

    eip = jnp.pad(edge_index, ((0, 0), (0, EP - EE)),
                  constant_values=NN).reshape(2, NW, KC, CH)

    eye8 = jnp.eye(8, dtype=_f32)
    w1n_bd = jnp.kron(eye8, W1n)
    w1s_bd = jnp.kron(eye8, W1s)
    w2n_bd = jnp.kron(eye8, W2n)
    w2s_bd = jnp.kron(eye8, W2s)
    b1t = jnp.tile(b1, 8).reshape(1, 128)
    b2t = jnp.tile(b2, 8).reshape(1, 128)
    wet = jnp.tile(We[:, 0], 8).reshape(1, 128)
    gs = jnp.kron(jnp.eye(8, dtype=_f32), jnp.ones((HH, 1), _f32))
    bet = be.reshape(1, 1)
    batp = jnp.pad(batch, (0, NP - NN),
                   constant_values=GG).reshape(RP, 8).T

    xp = jnp.pad(x, ((0, NP - NN), (0, 0))).reshape(RP, 8 * DD)
    y1n_p, ys1_p = pl.pallas_call(
        _proj_body,
        out_shape=(jax.ShapeDtypeStruct((RP, 128), _f32),
                   jax.ShapeDtypeStruct((RP, 128), _f32)),
    )(xp, w1n_bd, w1s_bd, b1t)

    agg1, deg1 = _edge_agg(y1n_p.reshape(NP, HH), eip, with_deg=True)

    y2n_p, ys2_p, degc_p = pl.pallas_call(
        _mid_body,
        out_shape=(jax.ShapeDtypeStruct((RP, 128), _f32),
                   jax.ShapeDtypeStruct((RP, 128), _f32),
                   jax.ShapeDtypeStruct((RP, 128), _f32)),
    )(agg1.reshape(NC, RP, 128), deg1.reshape(NC, RP, 128), ys1_p,
      w2n_bd, w2s_bd, b2t)

    (agg2,) = _edge_agg(y2n_p.reshape(NP, HH), eip, with_deg=False)

    out = pl.pallas_call(
        _final_body,
        out_shape=jax.ShapeDtypeStruct((GG, 1), _f32),
    )(agg2.reshape(NC, RP, 128), degc_p, ys2_p, wet, gs, bet, batp)
    return out


# speedup vs baseline: 37.5221x; 1.1097x over previous
import functools

import jax
import jax.numpy as jnp
from jax import lax
from jax.experimental import pallas as pl
from jax.experimental.pallas import tpu as pltpu
from jax.experimental.pallas import tpu_sc as plsc

NN = 10000
EE = 320000
DD = 128
HH = 16
GG = 64

NC = 2
NS = 16
NW = NC * NS
CH = 128
RB = 16
GA = 8
KC = 80
NI = KC // RB
EP = NW * KC * CH
NP = 10112
RT = NP // NS
RP = NP // 8

_f32 = jnp.float32


def _edge_agg_body(with_deg, *refs):
    if with_deg:
        (table_h, ei_h, acc_out, deg_out, srcv, dstv) = refs[:6]
        rows = refs[6:6 + RB]
        zst, ones, table_sh, acc_sh, deg_sh = refs[6 + RB:11 + RB]
        sems = refs[11 + RB:]
        gsem = sems[:GA]
        ssem = sems[GA:2 * GA]
        dsem = sems[2 * GA:3 * GA]
    else:
        (table_h, ei_h, acc_out, srcv, dstv) = refs[:5]
        rows = refs[5:5 + RB]
        zst, table_sh, acc_sh = refs[5 + RB:8 + RB]
        sems = refs[8 + RB:]
        gsem = sems[:GA]
        ssem = sems[GA:2 * GA]
        dsem = ones = None

    cid = lax.axis_index("c")
    sid = lax.axis_index("s")
    wid = sid * NC + cid

    zero16 = jnp.zeros((HH,), _f32)

    def zb(i, carry):
        zst[i, :] = zero16
        return carry

    lax.fori_loop(0, RT, zb, 0)

    if with_deg:
        one16 = jnp.ones((HH,), _f32)

        def ob(i, carry):
            ones[i, :] = one16
            return carry

        lax.fori_loop(0, CH, ob, 0)

    pltpu.sync_copy(zst, acc_sh.at[pl.ds(sid * RT, RT)])
    if with_deg:
        pltpu.sync_copy(zst, deg_sh.at[pl.ds(sid * RT, RT)])

    pltpu.sync_copy(table_h.at[pl.ds(sid * RT, RT)],
                    table_sh.at[pl.ds(sid * RT, RT)])

    pltpu.sync_copy(ei_h.at[0, wid], srcv)
    pltpu.sync_copy(ei_h.at[1, wid], dstv)

    plsc.subcore_barrier()

    def _drain(buf, sem):
        pltpu.make_async_copy(table_h.at[srcv.at[0]], buf, sem).wait()

    for p in range(GA):
        pltpu.async_copy(table_sh.at[srcv.at[p]], rows[p], gsem[p])

    def body(i, carry):
        for r in range(RB):
            j = RB * i + r
            p = r % GA
            pltpu.make_async_copy(
                table_h.at[srcv.at[0]], rows[r], gsem[p]).wait()
            if r < GA:
                @pl.when(i > 0)
                def _():
                    _drain(rows[(r + GA) % RB], ssem[p])
                    if with_deg:
                        _drain(rows[(r + GA) % RB], dsem[p])
            else:
                _drain(rows[(r + GA) % RB], ssem[p])
                if with_deg:
                    _drain(rows[(r + GA) % RB], dsem[p])
            if r < GA:
                pltpu.async_copy(
                    table_sh.at[srcv.at[j + GA]], rows[(r + GA) % RB],
                    gsem[p])
            else:
                @pl.when(i < NI - 1)
                def _():
                    pltpu.async_copy(
                        table_sh.at[srcv.at[j + GA]], rows[(r + GA) % RB],
                        gsem[p])
            pltpu.async_copy(rows[r], acc_sh.at[dstv.at[j]], ssem[p],
                             add=True)
            if with_deg:
                pltpu.async_copy(ones, deg_sh.at[dstv.at[j]], dsem[p],
                                 add=True)
        return carry

    lax.fori_loop(0, NI, body, 0)

    for p in range(GA):
        _drain(rows[GA + p], ssem[p])
        if with_deg:
            _drain(rows[GA + p], dsem[p])

    plsc.subcore_barrier()

    pltpu.sync_copy(acc_sh.at[pl.ds(sid * RT, RT)],
                    acc_out.at[cid, pl.ds(sid * RT, RT)])
    if with_deg:
        pltpu.sync_copy(deg_sh.at[pl.ds(sid * RT, RT)],
                        deg_out.at[cid, pl.ds(sid * RT, RT)])


def _edge_agg(table, eip, with_deg):
    out_type = [jax.ShapeDtypeStruct((NC, NP, HH), _f32)]
    scratch = [
        pltpu.VMEM((KC, CH), jnp.int32),
        pltpu.VMEM((KC, CH), jnp.int32),
    ]
    scratch += [pltpu.VMEM((CH, HH), _f32)] * RB
    scratch.append(pltpu.VMEM((RT, HH), _f32))
    if with_deg:
        out_type.append(jax.ShapeDtypeStruct((NC, NP, HH), _f32))
        scratch.append(pltpu.VMEM((CH, HH), _f32))
    scratch.append(pltpu.VMEM_SHARED((NP, HH), _f32))
    scratch.append(pltpu.VMEM_SHARED((NP, HH), _f32))
    if with_deg:
        scratch.append(pltpu.VMEM_SHARED((NP, HH), _f32))
    scratch.extend([pltpu.SemaphoreType.DMA] * (3 * GA if with_deg
                                                else 2 * GA))

    mesh = plsc.VectorSubcoreMesh(core_axis_name="c", subcore_axis_name="s")
    fn = pl.kernel(
        functools.partial(_edge_agg_body, with_deg),
        out_type=tuple(out_type),
        mesh=mesh,
        scratch_types=tuple(scratch),
        compiler_params=pltpu.CompilerParams(use_tc_tiling_on_sc=False),
    )
    return fn(table, eip)


def _proj_body(x_ref, wn_ref, ws_ref, b_ref, on_ref, os_ref):
    xv = x_ref[...]
    on_ref[...] = jnp.dot(xv, wn_ref[...], preferred_element_type=_f32)
    os_ref[...] = (jnp.dot(xv, ws_ref[...], preferred_element_type=_f32)
                   + b_ref[...])


def _mid_body(a_ref, d_ref, ys_ref, wn_ref, ws_ref, b_ref,
              on_ref, os_ref, dc_ref):
    deg = jnp.maximum(d_ref[0] + d_ref[1], 1.0)
    h1 = jnp.maximum((a_ref[0] + a_ref[1]) / deg + ys_ref[...], 0.0)
    on_ref[...] = jnp.dot(h1, wn_ref[...], preferred_element_type=_f32)
    os_ref[...] = (jnp.dot(h1, ws_ref[...], preferred_element_type=_f32)
                   + b_ref[...])
    dc_ref[...] = deg


def _final_body(a_ref, dc_ref, ys_ref, wet_ref, gs_ref, bet_ref, bat_ref,
                o_ref):
    h2 = jnp.maximum((a_ref[0] + a_ref[1]) / dc_ref[...] + ys_ref[...],
                     0.0)
    zw = h2 * wet_ref[...]
    zt = lax.dot_general(gs_ref[...], zw, (((0,), (1,)), ((), ())),
                         preferred_element_type=_f32)
    gid = lax.broadcasted_iota(jnp.int32, (GG, 8, RP), 0)
    m = (gid == bat_ref[...][None, :, :]).astype(_f32)
    s = jnp.sum(m * zt[None, :, :], axis=(1, 2))
    c = jnp.sum(m, axis=(1, 2))
    pooled = (s / jnp.maximum(c, 1.0))[:, None]
    pooled = pooled + jnp.where(c[:, None] > 0.0, bet_ref[...], 0.0)
    o_ref[...] = jax.nn.sigmoid(pooled)


def kernel(x, edge_index, edge_attr, batch, W1s, W1n, b1, W2s, W2n, b2,
           We, be):
    del edge_attr

    padi = NN + jnp.arange(EP - EE, dtype=jnp.int32) % (NP - NN)
    eip = jnp.concatenate(
        [edge_index, jnp.stack([padi, padi])], axis=1).reshape(
            2, NW, KC, CH)

    eye8 = jnp.eye(8, dtype=_f32)
    w1n_bd = jnp.kron(eye8, W1n)
    w1s_bd = jnp.kron(eye8, W1s)
    w2n_bd = jnp.kron(eye8, W2n)
    w2s_bd = jnp.kron(eye8, W2s)
    b1t = jnp.tile(b1, 8).reshape(1, 128)
    b2t = jnp.tile(b2, 8).reshape(1, 128)
    wet = jnp.tile(We[:, 0], 8).reshape(1, 128)
    gs = jnp.kron(jnp.eye(8, dtype=_f32), jnp.ones((HH, 1), _f32))
    bet = be.reshape(1, 1)
    batp = jnp.pad(batch, (0, NP - NN),
                   constant_values=GG).reshape(RP, 8).T

    xp = jnp.pad(x, ((0, NP - NN), (0, 0))).reshape(RP, 8 * DD)
    y1n_p, ys1_p = pl.pallas_call(
        _proj_body,
        out_shape=(jax.ShapeDtypeStruct((RP, 128), _f32),
                   jax.ShapeDtypeStruct((RP, 128), _f32)),
    )(xp, w1n_bd, w1s_bd, b1t)

    agg1, deg1 = _edge_agg(y1n_p.reshape(NP, HH), eip, with_deg=True)

    y2n_p, ys2_p, degc_p = pl.pallas_call(
        _mid_body,
        out_shape=(jax.ShapeDtypeStruct((RP, 128), _f32),
                   jax.ShapeDtypeStruct((RP, 128), _f32),
                   jax.ShapeDtypeStruct((RP, 128), _f32)),
    )(agg1.reshape(NC, RP, 128), deg1.reshape(NC, RP, 128), ys1_p,
      w2n_bd, w2s_bd, b2t)

    (agg2,) = _edge_agg(y2n_p.reshape(NP, HH), eip, with_deg=False)

    out = pl.pallas_call(
        _final_body,
        out_shape=jax.ShapeDtypeStruct((GG, 1), _f32),
    )(agg2.reshape(NC, RP, 128), degc_p, ys2_p, wet, gs, bet, batp)
    return out


# speedup vs baseline: 37.6982x; 1.0047x over previous
import functools

import jax
import jax.numpy as jnp
from jax import lax
from jax.experimental import pallas as pl
from jax.experimental.pallas import tpu as pltpu
from jax.experimental.pallas import tpu_sc as plsc

NN = 10000
EE = 320000
DD = 128
HH = 16
GG = 64

NC = 2
NS = 16
NW = NC * NS
CH = 125
RB = 16
GA = 8
KC = 80
NI = KC // RB
NP = 10112
RT = NP // NS
RP = NP // 8

_f32 = jnp.float32


def _edge_agg_body(with_deg, *refs):
    if with_deg:
        (table_h, ei_h, acc_out, deg_out, srcv, dstv) = refs[:6]
        rows = refs[6:6 + RB]
        zst, ones, table_sh, acc_sh, deg_sh = refs[6 + RB:11 + RB]
        sems = refs[11 + RB:]
        gsem = sems[:GA]
        ssem = sems[GA:2 * GA]
        dsem = sems[2 * GA:3 * GA]
    else:
        (table_h, ei_h, acc_out, srcv, dstv) = refs[:5]
        rows = refs[5:5 + RB]
        zst, table_sh, acc_sh = refs[5 + RB:8 + RB]
        sems = refs[8 + RB:]
        gsem = sems[:GA]
        ssem = sems[GA:2 * GA]
        dsem = ones = None

    cid = lax.axis_index("c")
    sid = lax.axis_index("s")
    wid = sid * NC + cid

    zero16 = jnp.zeros((HH,), _f32)

    def zb(i, carry):
        zst[i, :] = zero16
        return carry

    lax.fori_loop(0, RT, zb, 0)

    if with_deg:
        one16 = jnp.ones((HH,), _f32)

        def ob(i, carry):
            ones[i, :] = one16
            return carry

        lax.fori_loop(0, CH, ob, 0)

    pltpu.sync_copy(zst, acc_sh.at[pl.ds(sid * RT, RT)])
    if with_deg:
        pltpu.sync_copy(zst, deg_sh.at[pl.ds(sid * RT, RT)])

    pltpu.sync_copy(table_h.at[pl.ds(sid * RT, RT)],
                    table_sh.at[pl.ds(sid * RT, RT)])

    pltpu.sync_copy(ei_h.at[0, wid], srcv)
    pltpu.sync_copy(ei_h.at[1, wid], dstv)

    plsc.subcore_barrier()

    def _drain(buf, sem):
        pltpu.make_async_copy(table_h.at[srcv.at[0]], buf, sem).wait()

    for p in range(GA):
        pltpu.async_copy(table_sh.at[srcv.at[p]], rows[p], gsem[p])

    def body(i, carry):
        for r in range(RB):
            j = RB * i + r
            p = r % GA
            pltpu.make_async_copy(
                table_h.at[srcv.at[0]], rows[r], gsem[p]).wait()
            if r < GA:
                @pl.when(i > 0)
                def _():
                    _drain(rows[(r + GA) % RB], ssem[p])
                    if with_deg:
                        _drain(rows[(r + GA) % RB], dsem[p])
            else:
                _drain(rows[(r + GA) % RB], ssem[p])
                if with_deg:
                    _drain(rows[(r + GA) % RB], dsem[p])
            if r < GA:
                pltpu.async_copy(
                    table_sh.at[srcv.at[j + GA]], rows[(r + GA) % RB],
                    gsem[p])
            else:
                @pl.when(i < NI - 1)
                def _():
                    pltpu.async_copy(
                        table_sh.at[srcv.at[j + GA]], rows[(r + GA) % RB],
                        gsem[p])
            pltpu.async_copy(rows[r], acc_sh.at[dstv.at[j]], ssem[p],
                             add=True)
            if with_deg:
                pltpu.async_copy(ones, deg_sh.at[dstv.at[j]], dsem[p],
                                 add=True)
        return carry

    lax.fori_loop(0, NI, body, 0)

    for p in range(GA):
        _drain(rows[GA + p], ssem[p])
        if with_deg:
            _drain(rows[GA + p], dsem[p])

    plsc.subcore_barrier()

    pltpu.sync_copy(acc_sh.at[pl.ds(sid * RT, RT)],
                    acc_out.at[cid, pl.ds(sid * RT, RT)])
    if with_deg:
        pltpu.sync_copy(deg_sh.at[pl.ds(sid * RT, RT)],
                        deg_out.at[cid, pl.ds(sid * RT, RT)])


def _edge_agg(table, eip, with_deg):
    out_type = [jax.ShapeDtypeStruct((NC, NP, HH), _f32)]
    scratch = [
        pltpu.VMEM((KC, CH), jnp.int32),
        pltpu.VMEM((KC, CH), jnp.int32),
    ]
    scratch += [pltpu.VMEM((CH, HH), _f32)] * RB
    scratch.append(pltpu.VMEM((RT, HH), _f32))
    if with_deg:
        out_type.append(jax.ShapeDtypeStruct((NC, NP, HH), _f32))
        scratch.append(pltpu.VMEM((CH, HH), _f32))
    scratch.append(pltpu.VMEM_SHARED((NP, HH), _f32))
    scratch.append(pltpu.VMEM_SHARED((NP, HH), _f32))
    if with_deg:
        scratch.append(pltpu.VMEM_SHARED((NP, HH), _f32))
    scratch.extend([pltpu.SemaphoreType.DMA] * (3 * GA if with_deg
                                                else 2 * GA))

    mesh = plsc.VectorSubcoreMesh(core_axis_name="c", subcore_axis_name="s")
    fn = pl.kernel(
        functools.partial(_edge_agg_body, with_deg),
        out_type=tuple(out_type),
        mesh=mesh,
        scratch_types=tuple(scratch),
        compiler_params=pltpu.CompilerParams(use_tc_tiling_on_sc=False),
    )
    return fn(table, eip)


def _proj_body(x_ref, wn_ref, ws_ref, b_ref, on_ref, os_ref):
    xv = x_ref[...]
    on_ref[...] = jnp.dot(xv, wn_ref[...], preferred_element_type=_f32)
    os_ref[...] = (jnp.dot(xv, ws_ref[...], preferred_element_type=_f32)
                   + b_ref[...])


def _mid_body(a_ref, d_ref, ys_ref, wn_ref, ws_ref, b_ref,
              on_ref, os_ref, dc_ref):
    deg = jnp.maximum(d_ref[0] + d_ref[1], 1.0)
    h1 = jnp.maximum((a_ref[0] + a_ref[1]) / deg + ys_ref[...], 0.0)
    on_ref[...] = jnp.dot(h1, wn_ref[...], preferred_element_type=_f32)
    os_ref[...] = (jnp.dot(h1, ws_ref[...], preferred_element_type=_f32)
                   + b_ref[...])
    dc_ref[...] = deg


def _final_body(a_ref, dc_ref, ys_ref, wet_ref, gs_ref, bet_ref, bat_ref,
                o_ref):
    h2 = jnp.maximum((a_ref[0] + a_ref[1]) / dc_ref[...] + ys_ref[...],
                     0.0)
    zw = h2 * wet_ref[...]
    zt = lax.dot_general(gs_ref[...], zw, (((0,), (1,)), ((), ())),
                         preferred_element_type=_f32)
    gid = lax.broadcasted_iota(jnp.int32, (GG, 8, RP), 0)
    m = (gid == bat_ref[...][None, :, :]).astype(_f32)
    s = jnp.sum(m * zt[None, :, :], axis=(1, 2))
    c = jnp.sum(m, axis=(1, 2))
    pooled = (s / jnp.maximum(c, 1.0))[:, None]
    pooled = pooled + jnp.where(c[:, None] > 0.0, bet_ref[...], 0.0)
    o_ref[...] = jax.nn.sigmoid(pooled)


def kernel(x, edge_index, edge_attr, batch, W1s, W1n, b1, W2s, W2n, b2,
           We, be):
    del edge_attr

    eip = edge_index.reshape(2, NW, KC, CH)

    eye8 = jnp.eye(8, dtype=_f32)
    w1n_bd = jnp.kron(eye8, W1n)
    w1s_bd = jnp.kron(eye8, W1s)
    w2n_bd = jnp.kron(eye8, W2n)
    w2s_bd = jnp.kron(eye8, W2s)
    b1t = jnp.tile(b1, 8).reshape(1, 128)
    b2t = jnp.tile(b2, 8).reshape(1, 128)
    wet = jnp.tile(We[:, 0], 8).reshape(1, 128)
    gs = jnp.kron(jnp.eye(8, dtype=_f32), jnp.ones((HH, 1), _f32))
    bet = be.reshape(1, 1)
    batp = jnp.pad(batch, (0, NP - NN),
                   constant_values=GG).reshape(RP, 8).T

    xp = jnp.pad(x, ((0, NP - NN), (0, 0))).reshape(RP, 8 * DD)
    y1n_p, ys1_p = pl.pallas_call(
        _proj_body,
        out_shape=(jax.ShapeDtypeStruct((RP, 128), _f32),
                   jax.ShapeDtypeStruct((RP, 128), _f32)),
    )(xp, w1n_bd, w1s_bd, b1t)

    agg1, deg1 = _edge_agg(y1n_p.reshape(NP, HH), eip, with_deg=True)

    y2n_p, ys2_p, degc_p = pl.pallas_call(
        _mid_body,
        out_shape=(jax.ShapeDtypeStruct((RP, 128), _f32),
                   jax.ShapeDtypeStruct((RP, 128), _f32),
                   jax.ShapeDtypeStruct((RP, 128), _f32)),
    )(agg1.reshape(NC, RP, 128), deg1.reshape(NC, RP, 128), ys1_p,
      w2n_bd, w2s_bd, b2t)

    (agg2,) = _edge_agg(y2n_p.reshape(NP, HH), eip, with_deg=False)

    out = pl.pallas_call(
        _final_body,
        out_shape=jax.ShapeDtypeStruct((GG, 1), _f32),
    )(agg2.reshape(NC, RP, 128), degc_p, ys2_p, wet, gs, bet, batp)
    return out
